# baseline (speedup 1.0000x reference)
import jax
import jax.numpy as jnp
from jax.experimental import pallas as pl
from jax.experimental.pallas import tpu as pltpu

_E = 8
_R = 128
_D = 2048
_TILE = 1024


def _router_kernel(x_ref, w_ref, out_ref, wall_ref, rwt_ref):
    step = pl.program_id(0)
    nsteps = pl.num_programs(0)

    @pl.when(step == 0)
    def _pack_weights():
        for e in range(_E):
            wall_ref[:, e * _R:(e + 1) * _R] = w_ref[e]

    x = x_ref[...]
    y_all = jnp.dot(x, wall_ref[...],
                    preferred_element_type=jnp.float32)
    rows = []
    for e in range(_E):
        v = w_ref[e]
        y = y_all[:, e * _R:(e + 1) * _R]
        proj = jax.lax.dot_general(
            y, v, (((1,), (1,)), ((), ())),
            preferred_element_type=jnp.float32)
        dlt = x - proj
        rows.append(jnp.sum(dlt * dlt, axis=1))
    resid_t = jnp.sqrt(jnp.stack(rows, axis=0))
    neg = -resid_t
    neg = neg - jnp.max(neg, axis=0, keepdims=True)
    p = jnp.exp(neg)
    rw = p / jnp.sum(p, axis=0, keepdims=True)
    rwt_ref[:, pl.ds(step * _TILE, _TILE)] = rw

    @pl.when(step == nsteps - 1)
    def _epilogue():
        rw_all = rwt_ref[...]
        thr = 1.0 / _E
        row = jax.lax.broadcasted_iota(jnp.int32, rw_all.shape, 0)
        pre = rw_all > thr
        any_pre = jnp.any(pre)
        mx1 = jnp.max(rw_all, axis=0, keepdims=True)
        i1 = jnp.min(jnp.where(rw_all == mx1, row, _E), axis=0, keepdims=True)
        m1 = row == i1
        rw2 = jnp.where(m1, -jnp.float32(1.0), rw_all)
        mx2 = jnp.max(rw2, axis=0, keepdims=True)
        i2 = jnp.min(jnp.where(rw2 == mx2, row, _E), axis=0, keepdims=True)
        topk = m1 | (row == i2)
        mask_f = jnp.where(any_pre, pre.astype(jnp.float32),
                           m1.astype(jnp.float32)) * topk.astype(jnp.float32)
        filt = rw_all * mask_f
        s = jnp.sum(filt, axis=0, keepdims=True)
        s = jnp.where(s == 0.0, 1.0, s)
        out_ref[...] = (filt / s).T


def kernel(x, x_l, weight):
    del x_l
    tokens = x.shape[0]
    grid = (tokens // _TILE,)
    return pl.pallas_call(
        _router_kernel,
        grid=grid,
        in_specs=[
            pl.BlockSpec((_TILE, _D), lambda i: (i, 0)),
            pl.BlockSpec((_E, _D, _R), lambda i: (0, 0, 0)),
        ],
        out_specs=pl.BlockSpec((tokens, _E), lambda i: (0, 0)),
        out_shape=jax.ShapeDtypeStruct((tokens, _E), jnp.float32),
        scratch_shapes=[pltpu.VMEM((_D, _E * _R), jnp.float32),
                        pltpu.VMEM((_E, tokens), jnp.float32)],
        compiler_params=pltpu.CompilerParams(
            dimension_semantics=("arbitrary",)),
    )(x, weight)


# speedup vs baseline: 1.0462x; 1.0462x over previous
import jax
import jax.numpy as jnp
from jax.experimental import pallas as pl
from jax.experimental.pallas import tpu as pltpu

_E = 8
_R = 128
_D = 2048
_TILE = 1024


def _router_kernel(x_ref, w_ref, out_ref, wall_ref):
    step = pl.program_id(0)
    nsteps = pl.num_programs(0)

    @pl.when(step == 0)
    def _pack_weights():
        for e in range(_E):
            wall_ref[:, e * _R:(e + 1) * _R] = w_ref[e]

    x = x_ref[...]
    y_all = jnp.dot(x, wall_ref[...],
                    preferred_element_type=jnp.float32)
    rows = []
    for e in range(_E):
        v = w_ref[e]
        y = y_all[:, e * _R:(e + 1) * _R]
        proj = jax.lax.dot_general(
            y, v, (((1,), (1,)), ((), ())),
            preferred_element_type=jnp.float32)
        dlt = x - proj
        sq = dlt * dlt
        acc = sq[:, 0:128]
        for c in range(128, _D, 128):
            acc = acc + sq[:, c:c + 128]
        rows.append(jnp.sum(acc.T, axis=0))
    resid_t = jnp.sqrt(jnp.stack(rows, axis=0))
    neg = -resid_t
    neg = neg - jnp.max(neg, axis=0, keepdims=True)
    p = jnp.exp(neg)
    rw = p / jnp.sum(p, axis=0, keepdims=True)
    out_ref[:, pl.ds(step * _TILE, _TILE)] = rw

    @pl.when(step == nsteps - 1)
    def _epilogue():
        rw_all = out_ref[...]
        thr = 1.0 / _E
        row = jax.lax.broadcasted_iota(jnp.int32, rw_all.shape, 0)
        pre = rw_all > thr
        any_pre = jnp.any(pre)
        mx1 = jnp.max(rw_all, axis=0, keepdims=True)
        i1 = jnp.min(jnp.where(rw_all == mx1, row, _E), axis=0, keepdims=True)
        m1 = row == i1
        rw2 = jnp.where(m1, -jnp.float32(1.0), rw_all)
        mx2 = jnp.max(rw2, axis=0, keepdims=True)
        i2 = jnp.min(jnp.where(rw2 == mx2, row, _E), axis=0, keepdims=True)
        topk = m1 | (row == i2)
        mask_f = jnp.where(any_pre, pre.astype(jnp.float32),
                           m1.astype(jnp.float32)) * topk.astype(jnp.float32)
        filt = rw_all * mask_f
        s = jnp.sum(filt, axis=0, keepdims=True)
        s = jnp.where(s == 0.0, 1.0, s)
        out_ref[...] = filt / s


def kernel(x, x_l, weight):
    del x_l
    tokens = x.shape[0]
    grid = (tokens // _TILE,)
    out_t = pl.pallas_call(
        _router_kernel,
        grid=grid,
        in_specs=[
            pl.BlockSpec((_TILE, _D), lambda i: (i, 0)),
            pl.BlockSpec((_E, _D, _R), lambda i: (0, 0, 0)),
        ],
        out_specs=pl.BlockSpec((_E, tokens), lambda i: (0, 0)),
        out_shape=jax.ShapeDtypeStruct((_E, tokens), jnp.float32),
        scratch_shapes=[pltpu.VMEM((_D, _E * _R), jnp.float32)],
        compiler_params=pltpu.CompilerParams(
            dimension_semantics=("arbitrary",)),
    )(x, weight)
    return out_t.T
